# baseline (speedup 1.0000x reference)
import functools

import jax
import jax.numpy as jnp
import numpy as np
from jax import lax
from jax.experimental import pallas as pl
from jax.experimental.pallas import tpu as pltpu
from jax.experimental.pallas import tpu_sc as plsc

N_NODES_C = 4096
OUT_C = 64
HEADS_C = 4
HEAD_DIM_C = OUT_C // HEADS_C

NC = 2
NS = 16
LANES = 16
CD = 256
G = 128


SG = 1024
NFIRE = SG // G


def _sc_seg_body(n_nodes, n_edges, out_node, passes,
                 h_ref, a_ref, src_ref, dst_ref, out_ref,
                 src_a, dst_a, aw_a, rows_a, src_b, dst_b, aw_b, rows_b,
                 accs, sem_a, sem_b):
    c = lax.axis_index("c")
    s = lax.axis_index("s")
    hpt = out_node // LANES
    half = lax.rem(s, 2)
    pair = s // 2
    slab = pair * n_nodes
    nhalf = n_edges // 2
    nsg = nhalf // SG
    ebase = half * nhalf
    hn = n_nodes // 2

    for p in range(passes):
        chunk = p * NC * (NS // 2) + pair * NC + c
        head = chunk // hpt
        roff = chunk * n_nodes
        aoff = head * n_edges

        def zero_body(i, _):
            rows_a[pl.ds(i, 1)] = jnp.zeros((1, LANES), jnp.float32)
            return 0
        lax.fori_loop(0, SG, zero_body, 0)
        for i in range(hn // SG):
            pltpu.sync_copy(rows_a,
                            accs.at[pl.ds(slab + half * hn + i * SG, SG)])
        plsc.subcore_barrier()

        def fire(g, srcb, dstb, awb, rowsb, sem):
            eb = ebase + g * SG
            pltpu.sync_copy(src_ref.at[pl.ds(eb, SG)], srcb)
            pltpu.sync_copy(dst_ref.at[pl.ds(eb, SG)], dstb)
            pltpu.sync_copy(a_ref.at[pl.ds(aoff + eb, SG)], awb)

            def adj(k, _):
                sl = pl.ds(k * LANES, LANES)
                srcb[sl] = srcb[sl] + roff
                dstb[sl] = dstb[sl] + slab
                return 0
            lax.fori_loop(0, SG // LANES, adj, 0)
            for i in range(NFIRE):
                pltpu.async_copy(h_ref.at[srcb.at[pl.ds(i * G, G)]],
                                 rowsb.at[pl.ds(i * G, G)], sem)

        def wait(rowsb, sem):
            for i in range(NFIRE):
                pltpu.make_async_copy(h_ref.at[pl.ds(0, G)],
                                      rowsb.at[pl.ds(i * G, G)], sem).wait()

        def compute(dstb, awb, rowsb):
            def blk(e, _):
                rowsb[e] = rowsb[e] * awb[e]
                return 0
            lax.fori_loop(0, SG, blk, 0)
            pltpu.sync_copy(rowsb, accs.at[dstb], add=True)

        fire(0, src_a, dst_a, aw_a, rows_a, sem_a)

        def step(k, _):
            g0 = 2 * k
            wait(rows_a, sem_a)
            @pl.when(g0 + 1 < nsg)
            def _():
                fire(g0 + 1, src_b, dst_b, aw_b, rows_b, sem_b)
            compute(dst_a, aw_a, rows_a)
            @pl.when(g0 + 1 < nsg)
            def _():
                wait(rows_b, sem_b)
                @pl.when(g0 + 2 < nsg)
                def _():
                    fire(g0 + 2, src_a, dst_a, aw_a, rows_a, sem_a)
                compute(dst_b, aw_b, rows_b)
            return 0
        lax.fori_loop(0, (nsg + 1) // 2, step, 0)

        plsc.subcore_barrier()
        pltpu.sync_copy(
            accs.at[pl.ds(slab + half * hn, hn)],
            out_ref.at[pl.ds(chunk * n_nodes + half * hn, hn)])
        plsc.subcore_barrier()


def _sc_weighted_segment_sum(h, a, src, dst, n_nodes, heads, out_node):
    n_edges = src.shape[0]
    d = heads * out_node
    nchunk = d // LANES
    passes = nchunk // (NC * (NS // 2))
    h3 = h.reshape(n_nodes, nchunk, LANES).transpose(1, 0, 2).reshape(nchunk * n_nodes, LANES)
    a_t = jnp.broadcast_to(a.T.reshape(heads * n_edges, 1),
                           (heads * n_edges, LANES))
    mesh = plsc.VectorSubcoreMesh(core_axis_name="c", subcore_axis_name="s",
                                  num_cores=NC, num_subcores=NS)
    body = functools.partial(_sc_seg_body, n_nodes, n_edges, out_node, passes)
    out = pl.kernel(
        body,
        out_type=jax.ShapeDtypeStruct((nchunk * n_nodes, LANES), jnp.float32),
        mesh=mesh,
        compiler_params=pltpu.CompilerParams(use_tc_tiling_on_sc=False),
        scratch_types=[
            pltpu.VMEM((SG,), jnp.int32),
            pltpu.VMEM((SG,), jnp.int32),
            pltpu.VMEM((SG, LANES), jnp.float32),
            pltpu.VMEM((SG, LANES), jnp.float32),
            pltpu.VMEM((SG,), jnp.int32),
            pltpu.VMEM((SG,), jnp.int32),
            pltpu.VMEM((SG, LANES), jnp.float32),
            pltpu.VMEM((SG, LANES), jnp.float32),
            pltpu.VMEM_SHARED((NS // 2 * n_nodes, LANES), jnp.float32),
            pltpu.SemaphoreType.DMA,
            pltpu.SemaphoreType.DMA,
        ],
    )(h3, a_t, src, dst)
    return out.reshape(nchunk, n_nodes, LANES).transpose(1, 0, 2).reshape(n_nodes, d)


EG = 32


def _sc_edge_body(n_nodes, n_edges, d,
                  ni_ref, nj_ref, fij_ref, src_ref, dst_ref, out_ref,
                  src_st, dst_st, nia, nja, fa, nib, njb, fb,
                  sem_a, sem_b, sem_oa, sem_ob):
    c = lax.axis_index("c")
    s = lax.axis_index("s")
    wid = s * NC + c
    ep = n_edges // (NC * NS)
    nsg = ep // EG
    base = wid * ep

    pltpu.sync_copy(src_ref.at[pl.ds(base, ep)], src_st)
    pltpu.sync_copy(dst_ref.at[pl.ds(base, ep)], dst_st)

    def fire_in(g, nib_, njb_, fb_, sem):
        eb = g * EG
        pltpu.async_copy(ni_ref.at[src_st.at[pl.ds(eb, EG)]], nib_, sem)
        pltpu.async_copy(nj_ref.at[dst_st.at[pl.ds(eb, EG)]], njb_, sem)
        pltpu.async_copy(fij_ref.at[pl.ds(base + eb, EG)], fb_, sem)

    def wait_in(nib_, njb_, fb_, sem):
        pltpu.make_async_copy(ni_ref.at[pl.ds(0, EG)], nib_, sem).wait()
        pltpu.make_async_copy(nj_ref.at[pl.ds(0, EG)], njb_, sem).wait()
        pltpu.make_async_copy(fij_ref.at[pl.ds(0, EG)], fb_, sem).wait()

    def compute(nib_, njb_, fb_):
        def edge(e, _):
            for k in range(d // LANES):
                sl = pl.ds(k * LANES, LANES)
                t = nib_[e, sl] + njb_[e, sl] + fb_[e, sl]
                fb_[e, sl] = jnp.maximum(t, 0.0) + 0.01 * jnp.minimum(t, 0.0)
            return 0
        lax.fori_loop(0, EG, edge, 0)

    def fire_out(g, fb_, sem):
        pltpu.async_copy(fb_, out_ref.at[pl.ds(base + g * EG, EG)], sem)

    def wait_out(fb_, sem):
        pltpu.make_async_copy(fb_, out_ref.at[pl.ds(0, EG)], sem).wait()

    fire_in(0, nia, nja, fa, sem_a)
    fire_in(1, nib, njb, fb, sem_b)

    def step(k, _):
        g0 = 2 * k
        wait_in(nia, nja, fa, sem_a)
        compute(nia, nja, fa)
        fire_out(g0, fa, sem_oa)
        wait_in(nib, njb, fb, sem_b)
        @pl.when(g0 + 2 < nsg)
        def _():
            wait_out(fa, sem_oa)
            fire_in(g0 + 2, nia, nja, fa, sem_a)
        compute(nib, njb, fb)
        fire_out(g0 + 1, fb, sem_ob)
        @pl.when(g0 + 3 < nsg)
        def _():
            wait_out(fb, sem_ob)
            fire_in(g0 + 3, nib, njb, fb, sem_b)
        return 0
    lax.fori_loop(0, nsg // 2, step, 0)
    wait_out(fa, sem_oa)
    wait_out(fb, sem_ob)


def _sc_edge_combine(ni, nj, fij, src, dst):
    n_nodes, d = ni.shape
    n_edges = src.shape[0]
    mesh = plsc.VectorSubcoreMesh(core_axis_name="c", subcore_axis_name="s",
                                  num_cores=NC, num_subcores=NS)
    body = functools.partial(_sc_edge_body, n_nodes, n_edges, d)
    ep = n_edges // (NC * NS)
    out = pl.kernel(
        body,
        out_type=jax.ShapeDtypeStruct((n_edges, d), jnp.float32),
        mesh=mesh,
        scratch_types=[
            pltpu.VMEM((ep,), jnp.int32),
            pltpu.VMEM((ep,), jnp.int32),
            pltpu.VMEM((EG, d), jnp.float32),
            pltpu.VMEM((EG, d), jnp.float32),
            pltpu.VMEM((EG, d), jnp.float32),
            pltpu.VMEM((EG, d), jnp.float32),
            pltpu.VMEM((EG, d), jnp.float32),
            pltpu.VMEM((EG, d), jnp.float32),
            pltpu.SemaphoreType.DMA,
            pltpu.SemaphoreType.DMA,
            pltpu.SemaphoreType.DMA,
            pltpu.SemaphoreType.DMA,
        ],
    )(ni, nj, fij, src, dst)
    return out


def _attn_body(q_ref, k_ref, v_ref, o_ref, *, scale):
    q = q_ref[0]
    k = k_ref[0]
    v = v_ref[0]
    s = jax.lax.dot_general(q, k, (((1,), (1,)), ((), ())),
                            preferred_element_type=jnp.float32) * scale
    m = jnp.max(s, axis=-1, keepdims=True)
    p = jnp.exp(s - m)
    denom = jnp.sum(p, axis=-1, keepdims=True)
    o = jax.lax.dot_general(p, v, (((1,), (0,)), ((), ())),
                            preferred_element_type=jnp.float32)
    o_ref[0] = o / denom


def _global_attention(x, qkv_W, qkv_b, o_W, o_b):
    N = x.shape[0]
    qkv = x @ qkv_W.T + qkv_b
    qkv = qkv.reshape(N, HEADS_C, 3 * HEAD_DIM_C).transpose(1, 0, 2)
    q, k, v = jnp.split(qkv, 3, axis=-1)
    BQ = 512
    grid = (HEADS_C, N // BQ)
    out = pl.pallas_call(
        functools.partial(_attn_body, scale=1.0 / np.sqrt(HEAD_DIM_C)),
        grid=grid,
        in_specs=[
            pl.BlockSpec((1, BQ, HEAD_DIM_C), lambda h, i: (h, i, 0)),
            pl.BlockSpec((1, N, HEAD_DIM_C), lambda h, i: (h, 0, 0)),
            pl.BlockSpec((1, N, HEAD_DIM_C), lambda h, i: (h, 0, 0)),
        ],
        out_specs=pl.BlockSpec((1, BQ, HEAD_DIM_C), lambda h, i: (h, i, 0)),
        out_shape=jax.ShapeDtypeStruct((HEADS_C, N, HEAD_DIM_C), jnp.float32),
    )(q, k, v)
    vals = out.transpose(1, 0, 2).reshape(N, OUT_C)
    return vals @ o_W.T + o_b


def _edge_softmax(e, dst, n):
    m = jax.ops.segment_max(e, dst, num_segments=n)
    m = jnp.where(jnp.isfinite(m), m, 0.0)
    ex = jnp.exp(e - m[dst])
    s = jax.ops.segment_sum(ex, dst, num_segments=n)
    return ex / (s[dst] + 1e-9)


def _egat(p, nfeat, efeat, src, dst, n, heads, out_node, out_edge):
    f_ni = nfeat @ p['ni_W'].T
    f_nj = nfeat @ p['nj_W'].T
    f_fij = efeat @ p['fij_W'].T
    f_out = _sc_edge_combine(f_ni, f_nj, f_fij, src, dst).reshape(-1, heads, out_edge)
    e = jnp.sum(f_out * p['attn'], axis=-1, keepdims=True)
    a = _edge_softmax(e, dst, n)
    h = nfeat @ p['node_W'].T + p['node_b']
    h_out = _sc_weighted_segment_sum(h, a[:, :, 0], src, dst, n, heads, out_node)
    return h_out.reshape(n, heads, out_node), f_out


def kernel(all_node_data, all_edge_data, edge_index_g, edge_index_rq, n_r, params):
    p = params
    HEADS = 4
    OUT = 64
    N_LAYERS = 2
    src_g, dst_g = edge_index_g[0], edge_index_g[1]
    src_r, dst_r = edge_index_rq[0], edge_index_rq[1]
    N = all_node_data.shape[0]
    E1 = src_g.shape[0]
    node_data = all_node_data @ p['node_proj_W'].T + p['node_proj_b']
    edge_data = all_edge_data[:E1] @ p['edge_lin_W'].T + p['edge_lin_b']
    edge_data_link = all_edge_data[E1:] @ p['edge_lin_W'].T + p['edge_lin_b']
    for _ in range(N_LAYERS):
        h, e = _egat(p['conv1'], node_data, edge_data, src_g, dst_g, N, HEADS, 2 * OUT, 2 * OUT)
        h = jax.nn.relu(h)
        e = jax.nn.relu(e)
        h, e = _egat(p['conv2'], h.reshape(N, -1), e.reshape(E1, -1), src_g, dst_g, N, HEADS, 4 * OUT, OUT)
        edge_data = jax.nn.relu(e.reshape(E1, -1) @ p['lin1_W'].T + p['lin1_b'])
        node_data = jax.nn.relu(h.reshape(N, -1) @ p['lin2_W'].T + p['lin2_b'])
        h, e3 = _egat(p['rq'], node_data, edge_data_link, src_r, dst_r, N, HEADS, 2 * OUT, 2 * OUT)
        edge_data_link = jax.nn.relu(e3.reshape(e3.shape[0], -1) @ p['lin3_W'].T + p['lin3_b'])
        node_data = node_data + jax.nn.relu(h.reshape(N, -1) @ p['lin2a_W'].T + p['lin2a_b'])
    node_data = _global_attention(node_data, p['qkv_W'], p['qkv_b'], p['o_W'], p['o_b'])
    feat = jnp.max(node_data, axis=0)
    x2 = feat[None]
    x2 = jax.nn.relu(x2 @ p['reg1_W'].T + p['reg1_b'])
    x2 = jax.nn.relu(x2 @ p['reg2_W'].T + p['reg2_b'])
    loc = x2 @ p['reg3_W'].T + p['reg3_b']
    return (edge_data, node_data, loc)


# speedup vs baseline: 1.0884x; 1.0884x over previous
import functools

import jax
import jax.numpy as jnp
import numpy as np
from jax import lax
from jax.experimental import pallas as pl
from jax.experimental.pallas import tpu as pltpu
from jax.experimental.pallas import tpu_sc as plsc

N_NODES_C = 4096
OUT_C = 64
HEADS_C = 4
HEAD_DIM_C = OUT_C // HEADS_C

NC = 2
NS = 16
LANES = 16
CD = 256
G = 128


SG = 1024
NFIRE = SG // G


def _sc_seg_body(n_nodes, n_edges, out_node, passes,
                 h_ref, a_ref, src_ref, dst_ref, out_ref,
                 src_a, dst_a, aw_a, rows_a, src_b, dst_b, aw_b, rows_b,
                 acc, sem_a, sem_b):
    c = lax.axis_index("c")
    s = lax.axis_index("s")
    wid = s * NC + c
    hpt = out_node // LANES
    nsg = n_edges // SG

    for p in range(passes):
        chunk = p * NC * NS + wid
        head = chunk // hpt
        roff = chunk * n_nodes
        aoff = head * n_edges

        def zero_body(i, _):
            acc[i] = jnp.zeros((LANES,), jnp.float32)
            return 0
        lax.fori_loop(0, n_nodes, zero_body, 0)

        def fire(g, srcb, dstb, awb, rowsb, sem):
            eb = g * SG
            pltpu.sync_copy(src_ref.at[pl.ds(eb, SG)], srcb)
            pltpu.sync_copy(dst_ref.at[pl.ds(eb, SG)], dstb)
            pltpu.sync_copy(a_ref.at[pl.ds(aoff + eb, SG)], awb)

            def adj(k, _):
                sl = pl.ds(k * LANES, LANES)
                srcb[sl] = srcb[sl] + roff
                return 0
            lax.fori_loop(0, SG // LANES, adj, 0)
            for i in range(NFIRE):
                pltpu.async_copy(h_ref.at[srcb.at[pl.ds(i * G, G)]],
                                 rowsb.at[pl.ds(i * G, G)], sem)

        def wait(rowsb, sem):
            for i in range(NFIRE):
                pltpu.make_async_copy(h_ref.at[pl.ds(0, G)],
                                      rowsb.at[pl.ds(i * G, G)], sem).wait()

        def compute(dstb, awb, rowsb):
            def blk(b, _):
                dst_vec = dstb[pl.ds(b * LANES, LANES)]
                a_vec = awb[pl.ds(b * LANES, LANES)]
                for j in range(LANES):
                    e = b * LANES + j
                    plsc.addupdate(acc.at[dst_vec[j]], rowsb[e] * a_vec[j])
                return 0
            lax.fori_loop(0, SG // LANES, blk, 0)

        fire(0, src_a, dst_a, aw_a, rows_a, sem_a)

        def step(k, _):
            g0 = 2 * k
            wait(rows_a, sem_a)
            @pl.when(g0 + 1 < nsg)
            def _():
                fire(g0 + 1, src_b, dst_b, aw_b, rows_b, sem_b)
            compute(dst_a, aw_a, rows_a)
            @pl.when(g0 + 1 < nsg)
            def _():
                wait(rows_b, sem_b)
                @pl.when(g0 + 2 < nsg)
                def _():
                    fire(g0 + 2, src_a, dst_a, aw_a, rows_a, sem_a)
                compute(dst_b, aw_b, rows_b)
            return 0
        lax.fori_loop(0, (nsg + 1) // 2, step, 0)

        pltpu.sync_copy(acc, out_ref.at[pl.ds(chunk * n_nodes, n_nodes)])


def _sc_weighted_segment_sum(h, a, src, dst, n_nodes, heads, out_node):
    n_edges = src.shape[0]
    d = heads * out_node
    nchunk = d // LANES
    passes = nchunk // (NC * NS)
    h3 = h.reshape(n_nodes, nchunk, LANES).transpose(1, 0, 2).reshape(nchunk * n_nodes, LANES)
    a_t = a.T.reshape(-1)
    mesh = plsc.VectorSubcoreMesh(core_axis_name="c", subcore_axis_name="s",
                                  num_cores=NC, num_subcores=NS)
    body = functools.partial(_sc_seg_body, n_nodes, n_edges, out_node, passes)
    out = pl.kernel(
        body,
        out_type=jax.ShapeDtypeStruct((nchunk * n_nodes, LANES), jnp.float32),
        mesh=mesh,
        compiler_params=pltpu.CompilerParams(use_tc_tiling_on_sc=False),
        scratch_types=[
            pltpu.VMEM((SG,), jnp.int32),
            pltpu.VMEM((SG,), jnp.int32),
            pltpu.VMEM((SG,), jnp.float32),
            pltpu.VMEM((SG, LANES), jnp.float32),
            pltpu.VMEM((SG,), jnp.int32),
            pltpu.VMEM((SG,), jnp.int32),
            pltpu.VMEM((SG,), jnp.float32),
            pltpu.VMEM((SG, LANES), jnp.float32),
            pltpu.VMEM((n_nodes, LANES), jnp.float32),
            pltpu.SemaphoreType.DMA,
            pltpu.SemaphoreType.DMA,
        ],
    )(h3, a_t, src, dst)
    return out.reshape(nchunk, n_nodes, LANES).transpose(1, 0, 2).reshape(n_nodes, d)


EG = 32


def _sc_edge_body(n_nodes, n_edges, d,
                  ni_ref, nj_ref, fij_ref, src_ref, dst_ref, out_ref,
                  src_st, dst_st, nia, nja, fa, nib, njb, fb,
                  sem_a, sem_b, sem_oa, sem_ob):
    c = lax.axis_index("c")
    s = lax.axis_index("s")
    wid = s * NC + c
    ep = n_edges // (NC * NS)
    nsg = ep // EG
    base = wid * ep

    pltpu.sync_copy(src_ref.at[pl.ds(base, ep)], src_st)
    pltpu.sync_copy(dst_ref.at[pl.ds(base, ep)], dst_st)

    def fire_in(g, nib_, njb_, fb_, sem):
        eb = g * EG
        pltpu.async_copy(ni_ref.at[src_st.at[pl.ds(eb, EG)]], nib_, sem)
        pltpu.async_copy(nj_ref.at[dst_st.at[pl.ds(eb, EG)]], njb_, sem)
        pltpu.async_copy(fij_ref.at[pl.ds(base + eb, EG)], fb_, sem)

    def wait_in(nib_, njb_, fb_, sem):
        pltpu.make_async_copy(ni_ref.at[pl.ds(0, EG)], nib_, sem).wait()
        pltpu.make_async_copy(nj_ref.at[pl.ds(0, EG)], njb_, sem).wait()
        pltpu.make_async_copy(fij_ref.at[pl.ds(0, EG)], fb_, sem).wait()

    def compute(nib_, njb_, fb_):
        def edge(e, _):
            for k in range(d // LANES):
                sl = pl.ds(k * LANES, LANES)
                t = nib_[e, sl] + njb_[e, sl] + fb_[e, sl]
                fb_[e, sl] = jnp.maximum(t, 0.0) + 0.01 * jnp.minimum(t, 0.0)
            return 0
        lax.fori_loop(0, EG, edge, 0)

    def fire_out(g, fb_, sem):
        pltpu.async_copy(fb_, out_ref.at[pl.ds(base + g * EG, EG)], sem)

    def wait_out(fb_, sem):
        pltpu.make_async_copy(fb_, out_ref.at[pl.ds(0, EG)], sem).wait()

    fire_in(0, nia, nja, fa, sem_a)
    fire_in(1, nib, njb, fb, sem_b)

    def step(k, _):
        g0 = 2 * k
        wait_in(nia, nja, fa, sem_a)
        compute(nia, nja, fa)
        fire_out(g0, fa, sem_oa)
        wait_in(nib, njb, fb, sem_b)
        @pl.when(g0 + 2 < nsg)
        def _():
            wait_out(fa, sem_oa)
            fire_in(g0 + 2, nia, nja, fa, sem_a)
        compute(nib, njb, fb)
        fire_out(g0 + 1, fb, sem_ob)
        @pl.when(g0 + 3 < nsg)
        def _():
            wait_out(fb, sem_ob)
            fire_in(g0 + 3, nib, njb, fb, sem_b)
        return 0
    lax.fori_loop(0, nsg // 2, step, 0)
    wait_out(fa, sem_oa)
    wait_out(fb, sem_ob)


def _sc_edge_combine(ni, nj, fij, src, dst):
    n_nodes, d = ni.shape
    n_edges = src.shape[0]
    mesh = plsc.VectorSubcoreMesh(core_axis_name="c", subcore_axis_name="s",
                                  num_cores=NC, num_subcores=NS)
    body = functools.partial(_sc_edge_body, n_nodes, n_edges, d)
    ep = n_edges // (NC * NS)
    out = pl.kernel(
        body,
        out_type=jax.ShapeDtypeStruct((n_edges, d), jnp.float32),
        mesh=mesh,
        scratch_types=[
            pltpu.VMEM((ep,), jnp.int32),
            pltpu.VMEM((ep,), jnp.int32),
            pltpu.VMEM((EG, d), jnp.float32),
            pltpu.VMEM((EG, d), jnp.float32),
            pltpu.VMEM((EG, d), jnp.float32),
            pltpu.VMEM((EG, d), jnp.float32),
            pltpu.VMEM((EG, d), jnp.float32),
            pltpu.VMEM((EG, d), jnp.float32),
            pltpu.SemaphoreType.DMA,
            pltpu.SemaphoreType.DMA,
            pltpu.SemaphoreType.DMA,
            pltpu.SemaphoreType.DMA,
        ],
    )(ni, nj, fij, src, dst)
    return out


def _attn_body(q_ref, k_ref, v_ref, o_ref, *, scale):
    q = q_ref[0]
    k = k_ref[0]
    v = v_ref[0]
    s = jax.lax.dot_general(q, k, (((1,), (1,)), ((), ())),
                            preferred_element_type=jnp.float32) * scale
    m = jnp.max(s, axis=-1, keepdims=True)
    p = jnp.exp(s - m)
    denom = jnp.sum(p, axis=-1, keepdims=True)
    o = jax.lax.dot_general(p, v, (((1,), (0,)), ((), ())),
                            preferred_element_type=jnp.float32)
    o_ref[0] = o / denom


def _global_attention(x, qkv_W, qkv_b, o_W, o_b):
    N = x.shape[0]
    qkv = x @ qkv_W.T + qkv_b
    qkv = qkv.reshape(N, HEADS_C, 3 * HEAD_DIM_C).transpose(1, 0, 2)
    q, k, v = jnp.split(qkv, 3, axis=-1)
    BQ = 512
    grid = (HEADS_C, N // BQ)
    out = pl.pallas_call(
        functools.partial(_attn_body, scale=1.0 / np.sqrt(HEAD_DIM_C)),
        grid=grid,
        in_specs=[
            pl.BlockSpec((1, BQ, HEAD_DIM_C), lambda h, i: (h, i, 0)),
            pl.BlockSpec((1, N, HEAD_DIM_C), lambda h, i: (h, 0, 0)),
            pl.BlockSpec((1, N, HEAD_DIM_C), lambda h, i: (h, 0, 0)),
        ],
        out_specs=pl.BlockSpec((1, BQ, HEAD_DIM_C), lambda h, i: (h, i, 0)),
        out_shape=jax.ShapeDtypeStruct((HEADS_C, N, HEAD_DIM_C), jnp.float32),
    )(q, k, v)
    vals = out.transpose(1, 0, 2).reshape(N, OUT_C)
    return vals @ o_W.T + o_b


def _edge_softmax(e, dst, n):
    m = jax.ops.segment_max(e, dst, num_segments=n)
    m = jnp.where(jnp.isfinite(m), m, 0.0)
    ex = jnp.exp(e - m[dst])
    s = jax.ops.segment_sum(ex, dst, num_segments=n)
    return ex / (s[dst] + 1e-9)


def _egat(p, nfeat, efeat, src, dst, n, heads, out_node, out_edge):
    f_ni = nfeat @ p['ni_W'].T
    f_nj = nfeat @ p['nj_W'].T
    f_fij = efeat @ p['fij_W'].T
    f_out = _sc_edge_combine(f_ni, f_nj, f_fij, src, dst).reshape(-1, heads, out_edge)
    e = jnp.sum(f_out * p['attn'], axis=-1, keepdims=True)
    a = _edge_softmax(e, dst, n)
    h = nfeat @ p['node_W'].T + p['node_b']
    h_out = _sc_weighted_segment_sum(h, a[:, :, 0], src, dst, n, heads, out_node)
    return h_out.reshape(n, heads, out_node), f_out


def kernel(all_node_data, all_edge_data, edge_index_g, edge_index_rq, n_r, params):
    p = params
    HEADS = 4
    OUT = 64
    N_LAYERS = 2
    src_g, dst_g = edge_index_g[0], edge_index_g[1]
    src_r, dst_r = edge_index_rq[0], edge_index_rq[1]
    N = all_node_data.shape[0]
    E1 = src_g.shape[0]
    node_data = all_node_data @ p['node_proj_W'].T + p['node_proj_b']
    edge_data = all_edge_data[:E1] @ p['edge_lin_W'].T + p['edge_lin_b']
    edge_data_link = all_edge_data[E1:] @ p['edge_lin_W'].T + p['edge_lin_b']
    for _ in range(N_LAYERS):
        h, e = _egat(p['conv1'], node_data, edge_data, src_g, dst_g, N, HEADS, 2 * OUT, 2 * OUT)
        h = jax.nn.relu(h)
        e = jax.nn.relu(e)
        h, e = _egat(p['conv2'], h.reshape(N, -1), e.reshape(E1, -1), src_g, dst_g, N, HEADS, 4 * OUT, OUT)
        edge_data = jax.nn.relu(e.reshape(E1, -1) @ p['lin1_W'].T + p['lin1_b'])
        node_data = jax.nn.relu(h.reshape(N, -1) @ p['lin2_W'].T + p['lin2_b'])
        h, e3 = _egat(p['rq'], node_data, edge_data_link, src_r, dst_r, N, HEADS, 2 * OUT, 2 * OUT)
        edge_data_link = jax.nn.relu(e3.reshape(e3.shape[0], -1) @ p['lin3_W'].T + p['lin3_b'])
        node_data = node_data + jax.nn.relu(h.reshape(N, -1) @ p['lin2a_W'].T + p['lin2a_b'])
    node_data = _global_attention(node_data, p['qkv_W'], p['qkv_b'], p['o_W'], p['o_b'])
    feat = jnp.max(node_data, axis=0)
    x2 = feat[None]
    x2 = jax.nn.relu(x2 @ p['reg1_W'].T + p['reg1_b'])
    x2 = jax.nn.relu(x2 @ p['reg2_W'].T + p['reg2_b'])
    loc = x2 @ p['reg3_W'].T + p['reg3_b']
    return (edge_data, node_data, loc)
